# baseline (speedup 1.0000x reference)
import functools

import jax
import jax.numpy as jnp
from jax import lax
from jax.experimental import pallas as pl
from jax.experimental.pallas import tpu as pltpu
from jax.experimental.pallas import tpu_sc as plsc

B = 4
N = 2048
DIM = 128
K = 16
BN = B * N
BNK = BN * K

TCOL = 256


def _topk_body(boff, call_ref, crow_ref, fidx_ref):
    i = pl.program_id(0)
    call = call_ref[...]
    crow = crow_ref[...]
    dots = lax.dot_general(call, crow, (((1,), (0,)), ((), ())),
                           preferred_element_type=jnp.float32)
    sqj = jnp.sum(call * call, axis=1, keepdims=True)
    sqi = jnp.sum(crow * crow, axis=0, keepdims=True)
    d2 = sqj + sqi - 2.0 * dots
    dist = jnp.sqrt(jnp.maximum(d2, 0.0))
    sub = lax.broadcasted_iota(jnp.int32, (N, TCOL), 0)
    lane = lax.broadcasted_iota(jnp.int32, (N, TCOL), 1)
    dist = jnp.where(sub == lane + i * TCOL, dist + 1e6, dist)

    work = dist
    m = None
    for _ in range(K):
        m = jnp.min(work, axis=0, keepdims=True)
        work = jnp.where(work == m, jnp.float32(3e38), work)

    wt = lax.shift_left(jnp.int32(1), sub & 31)

    def pack(cond):
        bits = jnp.where(cond, wt, jnp.int32(0))
        return jnp.sum(bits.reshape(N // 32, 32, TCOL), axis=1,
                       dtype=jnp.int32)

    bm_s = pack(dist < m)
    bm_e = pack(dist == m)

    subbase = lax.broadcasted_iota(jnp.int32, (N // 32, TCOL), 0) * 32
    intmin = jnp.int32(-2147483648)

    def lowest(bm):
        low = bm & (jnp.int32(0) - bm)
        f = low.astype(jnp.float32)
        e = lax.shift_right_logical(
            lax.bitcast_convert_type(f, jnp.int32), 23) - 127
        e = jnp.where(low == intmin, jnp.int32(31), e)
        cand = jnp.where(bm != 0, subbase + e, jnp.int32(N))
        return cand, jnp.min(cand, axis=0, keepdims=True)

    picked = []
    for _ in range(K):
        cand_s, am_s = lowest(bm_s)
        cand_e, am_e = lowest(bm_e)
        pick = jnp.where(am_s < N, am_s, am_e)
        pick = jnp.minimum(pick, jnp.int32(N - 1))
        picked.append(pick)
        bm_s = jnp.where(cand_s == pick, bm_s & (bm_s - 1), bm_s)
        bm_e = jnp.where(cand_e == pick, bm_e & (bm_e - 1), bm_e)
    idxt = jnp.concatenate(picked, axis=0) + boff
    fidx_ref[...] = idxt


def _knn_topk_b(coords_b, coords_t_b, boff):
    return pl.pallas_call(
        functools.partial(_topk_body, boff),
        grid=(N // TCOL,),
        in_specs=[
            pl.BlockSpec((N, 3), lambda i: (0, 0)),
            pl.BlockSpec((3, TCOL), lambda i: (0, i)),
        ],
        out_specs=pl.BlockSpec((K, TCOL), lambda i: (0, i)),
        out_shape=jax.ShapeDtypeStruct((K, N), jnp.int32),
    )(coords_b, coords_t_b)


R_PQ = 1024


def _pq_body(x_ref, wp_ref, wq_ref, b1_ref, p_ref, q_ref):
    x = x_ref[...]
    p_ref[...] = lax.dot_general(x, wp_ref[...], (((1,), (0,)), ((), ())),
                                 preferred_element_type=jnp.float32) + b1_ref[...]
    q_ref[...] = lax.dot_general(x, wq_ref[...], (((1,), (0,)), ((), ())),
                                 preferred_element_type=jnp.float32)


def _pq(x, wp, wq, b1r):
    return pl.pallas_call(
        _pq_body,
        grid=(BN // R_PQ,),
        in_specs=[
            pl.BlockSpec((R_PQ, DIM + 3), lambda i: (i, 0)),
            pl.BlockSpec((DIM + 3, DIM), lambda i: (0, 0)),
            pl.BlockSpec((DIM + 3, DIM), lambda i: (0, 0)),
            pl.BlockSpec((1, DIM), lambda i: (0, 0)),
        ],
        out_specs=[
            pl.BlockSpec((R_PQ, DIM), lambda i: (i, 0)),
            pl.BlockSpec((R_PQ, DIM), lambda i: (i, 0)),
        ],
        out_shape=[
            jax.ShapeDtypeStruct((BN, DIM), jnp.float32),
            jax.ShapeDtypeStruct((BN, DIM), jnp.float32),
        ],
    )(x, wp, wq, b1r)


SC_NC = 2
SC_NS = 16
SC_NW = SC_NC * SC_NS


def _sc_gather_body(rows_per_w, idx_hbm, q_hbm, out_hbm, idx_v, rows_v, sem):
    wid = lax.axis_index("s") * SC_NC + lax.axis_index("c")
    base = wid * rows_per_w
    pltpu.sync_copy(idx_hbm.at[pl.ds(base, rows_per_w)], idx_v)

    def body(r, carry):
        pltpu.async_copy(q_hbm.at[idx_v.at[r]], rows_v, sem).wait()
        pltpu.sync_copy(rows_v, out_hbm.at[pl.ds((base + r) * 128, 128)])
        return carry

    lax.fori_loop(0, rows_per_w, body, 0)


@functools.cache
def _sc_gather_kernel(nidx):
    rows_per_w = nidx // 128 // SC_NW
    return pl.kernel(
        functools.partial(_sc_gather_body, rows_per_w),
        mesh=plsc.VectorSubcoreMesh(core_axis_name="c", subcore_axis_name="s"),
        out_type=jax.ShapeDtypeStruct((nidx, DIM), jnp.float32),
        scratch_types=[
            pltpu.VMEM((rows_per_w, 128), jnp.int32),
            pltpu.VMEM((128, DIM), jnp.float32),
            pltpu.SemaphoreType.DMA,
        ],
    )


def _sc_gather(idx2, q):
    return _sc_gather_kernel(idx2.shape[0] * 128)(idx2, q)


R_MLP = 128


def _gelu_exact(x):
    return 0.5 * x * (1.0 + lax.erf(x * jnp.float32(0.7071067811865476)))


def _mlp_body(qg_ref, p_ref, w2_ref, b2_ref, out_ref):
    x = qg_ref[...]
    pblk = p_ref[...]
    h = x + pblk[None, :, :]
    h = _gelu_exact(h)
    h2 = lax.dot_general(h.reshape(K * R_MLP, DIM), w2_ref[...],
                         (((1,), (0,)), ((), ())),
                         preferred_element_type=jnp.float32) + b2_ref[...]
    z = h2.reshape(K, R_MLP, DIM)
    zmax = jnp.max(z, axis=0)
    zmin = jnp.min(z, axis=0)
    out_ref[...] = jnp.maximum(_gelu_exact(zmax), _gelu_exact(zmin))


def _mlp(qg3, p, w2, b2r):
    n = p.shape[0]
    return pl.pallas_call(
        _mlp_body,
        grid=(n // R_MLP,),
        in_specs=[
            pl.BlockSpec((K, R_MLP, DIM), lambda i: (0, i, 0)),
            pl.BlockSpec((R_MLP, DIM), lambda i: (i, 0)),
            pl.BlockSpec((DIM, DIM), lambda i: (0, 0)),
            pl.BlockSpec((1, DIM), lambda i: (0, 0)),
        ],
        out_specs=pl.BlockSpec((R_MLP, DIM), lambda i: (i, 0)),
        out_shape=jax.ShapeDtypeStruct((n, DIM), jnp.float32),
    )(qg3, p, w2, b2r)


def kernel(feats, coords, W1, b1, W2, b2):
    coords_t = jnp.swapaxes(coords, 1, 2)

    wa = W1[:DIM]
    wb = W1[DIM:2 * DIM]
    wc = W1[2 * DIM:]
    wp = jnp.concatenate([wa - wb, -wc], axis=0)
    wq = jnp.concatenate([wb, wc], axis=0)
    x = jnp.concatenate(
        [feats.reshape(BN, DIM), coords.reshape(BN, 3)], axis=1)
    p, q = _pq(x, wp, wq, b1.reshape(1, DIM))

    b2r = b2.reshape(1, DIM)
    outs = []
    qgs = []
    for b in range(B):
        fidx_b = _knn_topk_b(coords[b], coords_t[b], b * N)
        qgs.append(_sc_gather(fidx_b.reshape(N * K // 128, 128), q))
    for b in range(B):
        outs.append(_mlp(qgs[b].reshape(K, N, DIM),
                         p[b * N:(b + 1) * N], W2, b2r))
    return jnp.stack(outs).reshape(B, N, DIM)


# speedup vs baseline: 1.7697x; 1.7697x over previous
import functools

import jax
import jax.numpy as jnp
from jax import lax
from jax.experimental import pallas as pl
from jax.experimental.pallas import tpu as pltpu
from jax.experimental.pallas import tpu_sc as plsc

B = 4
N = 2048
DIM = 128
K = 16
BN = B * N
BNK = BN * K

TCOL = 256


def _topk_body(boff, call_ref, crow_ref, fidx_ref):
    i = pl.program_id(0)
    call = call_ref[...]
    crow = crow_ref[...]
    dots = lax.dot_general(call, crow, (((1,), (0,)), ((), ())),
                           preferred_element_type=jnp.float32)
    sqj = jnp.sum(call * call, axis=1, keepdims=True)
    sqi = jnp.sum(crow * crow, axis=0, keepdims=True)
    d2 = sqj + sqi - 2.0 * dots
    dist = jnp.sqrt(jnp.maximum(d2, 0.0))
    sub = lax.broadcasted_iota(jnp.int32, (N, TCOL), 0)
    lane = lax.broadcasted_iota(jnp.int32, (N, TCOL), 1)
    dist = jnp.where(sub == lane + i * TCOL, dist + 1e6, dist)

    work = dist
    picked = []
    for _ in range(K):
        m = jnp.min(work, axis=0, keepdims=True)
        idxm = jnp.where(work == m, sub, jnp.int32(N))
        am = jnp.min(idxm, axis=0, keepdims=True)
        picked.append(am)
        work = jnp.where(sub == am, jnp.float32(3e38), work)
    idxt = jnp.concatenate(picked, axis=0) + boff
    fidx_ref[...] = idxt


def _knn_topk_b(coords_b, coords_t_b, boff):
    return pl.pallas_call(
        functools.partial(_topk_body, boff),
        grid=(N // TCOL,),
        in_specs=[
            pl.BlockSpec((N, 3), lambda i: (0, 0)),
            pl.BlockSpec((3, TCOL), lambda i: (0, i)),
        ],
        out_specs=pl.BlockSpec((K, TCOL), lambda i: (0, i)),
        out_shape=jax.ShapeDtypeStruct((K, N), jnp.int32),
    )(coords_b, coords_t_b)


R_PQ = 1024


def _pq_body(x_ref, wp_ref, wq_ref, b1_ref, p_ref, q_ref):
    x = x_ref[...]
    p_ref[...] = lax.dot_general(x, wp_ref[...], (((1,), (0,)), ((), ())),
                                 preferred_element_type=jnp.float32) + b1_ref[...]
    q_ref[...] = lax.dot_general(x, wq_ref[...], (((1,), (0,)), ((), ())),
                                 preferred_element_type=jnp.float32)


def _pq(x, wp, wq, b1r):
    return pl.pallas_call(
        _pq_body,
        grid=(BN // R_PQ,),
        in_specs=[
            pl.BlockSpec((R_PQ, DIM + 3), lambda i: (i, 0)),
            pl.BlockSpec((DIM + 3, DIM), lambda i: (0, 0)),
            pl.BlockSpec((DIM + 3, DIM), lambda i: (0, 0)),
            pl.BlockSpec((1, DIM), lambda i: (0, 0)),
        ],
        out_specs=[
            pl.BlockSpec((R_PQ, DIM), lambda i: (i, 0)),
            pl.BlockSpec((R_PQ, DIM), lambda i: (i, 0)),
        ],
        out_shape=[
            jax.ShapeDtypeStruct((BN, DIM), jnp.float32),
            jax.ShapeDtypeStruct((BN, DIM), jnp.float32),
        ],
    )(x, wp, wq, b1r)


SC_NC = 2
SC_NS = 16
SC_NW = SC_NC * SC_NS


def _sc_gather_body(rows_per_w, idx_hbm, q_hbm, out_hbm, idx_v, rows_v, sem):
    wid = lax.axis_index("s") * SC_NC + lax.axis_index("c")
    base = wid * rows_per_w
    pltpu.sync_copy(idx_hbm.at[pl.ds(base, rows_per_w)], idx_v)

    def body(r, carry):
        pltpu.async_copy(q_hbm.at[idx_v.at[r]], rows_v, sem).wait()
        pltpu.sync_copy(rows_v, out_hbm.at[pl.ds((base + r) * 128, 128)])
        return carry

    lax.fori_loop(0, rows_per_w, body, 0)


@functools.cache
def _sc_gather_kernel(nidx):
    rows_per_w = nidx // 128 // SC_NW
    return pl.kernel(
        functools.partial(_sc_gather_body, rows_per_w),
        mesh=plsc.VectorSubcoreMesh(core_axis_name="c", subcore_axis_name="s"),
        out_type=jax.ShapeDtypeStruct((nidx, DIM), jnp.float32),
        scratch_types=[
            pltpu.VMEM((rows_per_w, 128), jnp.int32),
            pltpu.VMEM((128, DIM), jnp.float32),
            pltpu.SemaphoreType.DMA,
        ],
    )


def _sc_gather(idx2, q):
    return _sc_gather_kernel(idx2.shape[0] * 128)(idx2, q)


R_MLP = 128


def _gelu_exact(x):
    return 0.5 * x * (1.0 + lax.erf(x * jnp.float32(0.7071067811865476)))


def _mlp_body(qg_ref, p_ref, w2_ref, b2_ref, out_ref):
    x = qg_ref[...]
    pblk = p_ref[...]
    h = x + pblk[None, :, :]
    h = _gelu_exact(h)
    h2 = lax.dot_general(h.reshape(K * R_MLP, DIM), w2_ref[...],
                         (((1,), (0,)), ((), ())),
                         preferred_element_type=jnp.float32) + b2_ref[...]
    z = h2.reshape(K, R_MLP, DIM)
    zmax = jnp.max(z, axis=0)
    zmin = jnp.min(z, axis=0)
    out_ref[...] = jnp.maximum(_gelu_exact(zmax), _gelu_exact(zmin))


def _mlp(qg3, p, w2, b2r):
    n = p.shape[0]
    return pl.pallas_call(
        _mlp_body,
        grid=(n // R_MLP,),
        in_specs=[
            pl.BlockSpec((K, R_MLP, DIM), lambda i: (0, i, 0)),
            pl.BlockSpec((R_MLP, DIM), lambda i: (i, 0)),
            pl.BlockSpec((DIM, DIM), lambda i: (0, 0)),
            pl.BlockSpec((1, DIM), lambda i: (0, 0)),
        ],
        out_specs=pl.BlockSpec((R_MLP, DIM), lambda i: (i, 0)),
        out_shape=jax.ShapeDtypeStruct((n, DIM), jnp.float32),
    )(qg3, p, w2, b2r)


def kernel(feats, coords, W1, b1, W2, b2):
    coords_t = jnp.swapaxes(coords, 1, 2)

    wa = W1[:DIM]
    wb = W1[DIM:2 * DIM]
    wc = W1[2 * DIM:]
    wp = jnp.concatenate([wa - wb, -wc], axis=0)
    wq = jnp.concatenate([wb, wc], axis=0)
    x = jnp.concatenate(
        [feats.reshape(BN, DIM), coords.reshape(BN, 3)], axis=1)
    p, q = _pq(x, wp, wq, b1.reshape(1, DIM))

    b2r = b2.reshape(1, DIM)
    outs = []
    qgs = []
    for b in range(B):
        fidx_b = _knn_topk_b(coords[b], coords_t[b], b * N)
        qgs.append(_sc_gather(fidx_b.reshape(N * K // 128, 128), q))
    for b in range(B):
        outs.append(_mlp(qgs[b].reshape(K, N, DIM),
                         p[b * N:(b + 1) * N], W2, b2r))
    return jnp.stack(outs).reshape(B, N, DIM)
